# baseline (speedup 1.0000x reference)
import functools
import math

import jax
import jax.numpy as jnp
from jax import lax
from jax.experimental import pallas as pl
from jax.experimental.pallas import tpu as pltpu
from jax.experimental.pallas import tpu_sc as plsc

_INT_MIN = -2147483648


def _skey(f):
    i = jax.lax.bitcast_convert_type(f, jnp.int32)
    return jnp.where(i >= 0, i, jnp.int32(_INT_MIN) - i)


def _kth_largest_key(skey, k, count_fn, row_shape):
    K = jnp.full(row_shape, _INT_MIN, dtype=jnp.int32)
    for b in range(31, -1, -1):
        addend = _INT_MIN if b == 31 else (1 << b)
        cand = K + jnp.int32(addend)
        cnt = count_fn(skey >= _bcast_row(cand, skey.shape))
        K = jnp.where(cnt >= k, cand, K)
    return K


def _bcast_row(row_val, full_shape):
    return jnp.broadcast_to(row_val[..., None], full_shape)


def _make_tc_kernel(k1, k2):
    def _kern(x_ref, o_ref):
        x = x_ref[...]
        s1 = _skey(x)

        def count1(ge):
            return jnp.sum(ge.astype(jnp.int32), axis=2)

        K1 = _kth_largest_key(s1, k1, count1, s1.shape[:2])
        keep1 = s1 >= _bcast_row(K1, s1.shape)
        m1 = jnp.where(keep1, x, 0.0)
        s2 = jnp.where(keep1, s1, 0)

        def count2(ge):
            c = jnp.sum(ge.astype(jnp.int32), axis=2)
            return jnp.sum(c, axis=0, keepdims=True)

        K2 = _kth_largest_key(s2, k2, count2, (1, s1.shape[1]))
        K2 = jnp.broadcast_to(K2, s1.shape[:2])
        o_ref[...] = jnp.where(s2 >= _bcast_row(K2, s1.shape), m1, 0.0)

    return _kern


def _tc_topk(xr, k1, k2):
    B, C, HW = xr.shape
    cb = C
    for c_try in (64, 32, 16, 8, 4, 2, 1):
        if C % c_try == 0:
            cb = c_try
            break
    return pl.pallas_call(
        _make_tc_kernel(k1, k2),
        grid=(C // cb,),
        in_specs=[pl.BlockSpec((B, cb, HW), lambda i: (0, i, 0))],
        out_specs=pl.BlockSpec((B, cb, HW), lambda i: (0, i, 0)),
        out_shape=jax.ShapeDtypeStruct((B, C, HW), jnp.float32),
    )(xr)


_NC, _NS, _L = 2, 16, 16


def _splat_i32(x):
    return jnp.broadcast_to(jnp.asarray(x, jnp.int32), (_L,))


def _foreach16(n16, fn, init):
    unroll = next(u for u in (8, 4, 2, 1) if n16 % u == 0)

    def body(j, carry):
        for u in range(unroll):
            carry = fn(j * unroll + u, carry)
        return carry

    return lax.fori_loop(0, n16 // unroll, body, init)


def _lane_sum(cnt):
    total = cnt[0]
    for i in range(1, _L):
        total = total + cnt[i]
    return total


def _kth_key_sc(read_slice, n16, k):
    def bit_body(bi, K):
        shift = jnp.broadcast_to(31 - bi, (_L,)).astype(jnp.int32)
        cand = K + (_splat_i32(1) << shift)

        def cnt_fn(i, cnt):
            ge = read_slice(i) >= cand
            return cnt + jnp.where(ge, _splat_i32(1), _splat_i32(0))

        cnt = _foreach16(n16, cnt_fn, _splat_i32(0))
        return jnp.where(_lane_sum(cnt) >= k, cand, K)

    return lax.fori_loop(0, 32, bit_body, _splat_i32(_INT_MIN))


def _make_sc_kernel(B, HW, csc, k1, k2):
    n16_map = HW // _L
    n16_ch = (B * HW) // _L
    nw = _NC * _NS
    cpw = csc // nw
    mesh = plsc.VectorSubcoreMesh(
        core_axis_name="c", subcore_axis_name="s",
        num_cores=_NC, num_subcores=_NS)

    @functools.partial(
        pl.kernel,
        mesh=mesh,
        out_type=jax.ShapeDtypeStruct((B, csc, HW), jnp.float32),
        scratch_types=[
            pltpu.VMEM((B * HW,), jnp.float32),
            pltpu.VMEM((B * HW,), jnp.int32),
            pltpu.SemaphoreType.DMA,
        ],
    )
    def sck(x_hbm, o_hbm, xv, sv, sem):
        wid = lax.axis_index("s") * _NC + lax.axis_index("c")

        def chan_body(ci, _):
            ch = wid * cpw + ci
            cps = [
                pltpu.async_copy(x_hbm.at[b, ch], xv.at[pl.ds(b * HW, HW)], sem)
                for b in range(B)
            ]
            for cp in cps:
                cp.wait()

            def key_fn(i, carry):
                idx = pl.ds(i * _L, _L)
                iv = lax.bitcast_convert_type(xv[idx], jnp.int32)
                sv[idx] = jnp.where(iv >= 0, iv, _splat_i32(_INT_MIN) - iv)
                return carry

            _foreach16(n16_ch, key_fn, 0)

            def map_body(m, _):
                base = m * n16_map

                K1 = _kth_key_sc(
                    lambda i: sv[pl.ds((base + i) * _L, _L)], n16_map, k1)

                def mask_fn(i, carry):
                    idx = pl.ds((base + i) * _L, _L)
                    s = sv[idx]
                    keep = s >= K1
                    xv[idx] = jnp.where(keep, xv[idx], 0.0)
                    sv[idx] = jnp.where(keep, s, _splat_i32(0))
                    return carry

                _foreach16(n16_map, mask_fn, 0)
                return 0

            lax.fori_loop(0, B, map_body, 0)

            K2 = _kth_key_sc(lambda i: sv[pl.ds(i * _L, _L)], n16_ch, k2)

            def mask2_fn(i, carry):
                idx = pl.ds(i * _L, _L)
                xv[idx] = jnp.where(sv[idx] >= K2, xv[idx], 0.0)
                return carry

            _foreach16(n16_ch, mask2_fn, 0)

            cps_o = [
                pltpu.async_copy(xv.at[pl.ds(b * HW, HW)], o_hbm.at[b, ch], sem)
                for b in range(B)
            ]
            for cp in cps_o:
                cp.wait()
            return 0

        lax.fori_loop(0, cpw, chan_body, 0)

    return sck


_C_SC = 384


def kernel(x):
    B, C, H, W = x.shape
    HW = H * W
    k1 = math.ceil(0.5 * H * W)
    k2 = math.ceil(0.5 * k1 * B)
    xr = x.reshape(B, C, HW)

    csc = min(_C_SC, C)
    csc -= csc % (_NC * _NS)
    parts = []
    if csc:
        parts.append(_make_sc_kernel(B, HW, csc, k1, k2)(xr[:, :csc]))
    if csc < C:
        parts.append(_tc_topk(xr[:, csc:], k1, k2))
    out = parts[0] if len(parts) == 1 else jnp.concatenate(parts, axis=1)
    return out.reshape(B, C, H, W)


# speedup vs baseline: 2.4601x; 2.4601x over previous
import functools
import math

import jax
import jax.numpy as jnp
from jax import lax
from jax.experimental import pallas as pl
from jax.experimental.pallas import tpu as pltpu
from jax.experimental.pallas import tpu_sc as plsc

_INT_MIN = -2147483648


def _skey(f):
    i = jax.lax.bitcast_convert_type(f, jnp.int32)
    return jnp.where(i >= 0, i, jnp.int32(_INT_MIN) - i)


def _kth_largest_key(skey, k, count_fn, row_shape):
    K = jnp.full(row_shape, _INT_MIN, dtype=jnp.int32)
    for b in range(31, -1, -1):
        addend = _INT_MIN if b == 31 else (1 << b)
        cand = K + jnp.int32(addend)
        cnt = count_fn(skey >= _bcast_row(cand, skey.shape))
        K = jnp.where(cnt >= k, cand, K)
    return K


def _bcast_row(row_val, full_shape):
    return jnp.broadcast_to(row_val[..., None], full_shape)


def _make_tc_kernel(k1, k2):
    def _kern(x_ref, o_ref):
        x = x_ref[...]
        s1 = _skey(x)

        def count1(ge):
            return jnp.sum(ge.astype(jnp.int32), axis=2)

        K1 = _kth_largest_key(s1, k1, count1, s1.shape[:2])
        keep1 = s1 >= _bcast_row(K1, s1.shape)
        m1 = jnp.where(keep1, x, 0.0)
        s2 = jnp.where(keep1, s1, 0)

        def count2(ge):
            c = jnp.sum(ge.astype(jnp.int32), axis=2)
            return jnp.sum(c, axis=0, keepdims=True)

        K2 = _kth_largest_key(s2, k2, count2, (1, s1.shape[1]))
        K2 = jnp.broadcast_to(K2, s1.shape[:2])
        o_ref[...] = jnp.where(s2 >= _bcast_row(K2, s1.shape), m1, 0.0)

    return _kern


def _tc_topk(xr, k1, k2):
    B, C, HW = xr.shape
    cb = C
    for c_try in (64, 32, 16, 8, 4, 2, 1):
        if C % c_try == 0:
            cb = c_try
            break
    return pl.pallas_call(
        _make_tc_kernel(k1, k2),
        grid=(C // cb,),
        in_specs=[pl.BlockSpec((B, cb, HW), lambda i: (0, i, 0))],
        out_specs=pl.BlockSpec((B, cb, HW), lambda i: (0, i, 0)),
        out_shape=jax.ShapeDtypeStruct((B, C, HW), jnp.float32),
    )(xr)


_NC, _NS, _L = 2, 16, 16


def _splat_i32(x):
    return jnp.broadcast_to(jnp.asarray(x, jnp.int32), (_L,))


def _foreach16(n16, fn, init):
    unroll = next(u for u in (8, 4, 2, 1) if n16 % u == 0)

    def body(j, carry):
        for u in range(unroll):
            carry = fn(j * unroll + u, carry)
        return carry

    return lax.fori_loop(0, n16 // unroll, body, init)


def _lane_sum(cnt):
    total = cnt[0]
    for i in range(1, _L):
        total = total + cnt[i]
    return total


def _kth_key_sc(read_slice, n16, k):
    def bit_body(bi, K):
        shift = jnp.broadcast_to(31 - bi, (_L,)).astype(jnp.int32)
        cand = K + (_splat_i32(1) << shift)

        def cnt_fn(i, cnt):
            ge = read_slice(i) >= cand
            return cnt + jnp.where(ge, _splat_i32(1), _splat_i32(0))

        cnt = _foreach16(n16, cnt_fn, _splat_i32(0))
        return jnp.where(_lane_sum(cnt) >= k, cand, K)

    return lax.fori_loop(0, 32, bit_body, _splat_i32(_INT_MIN))


def _make_sc_kernel(B, HW, csc, k1, k2):
    n16_map = HW // _L
    n16_ch = (B * HW) // _L
    nw = _NC * _NS
    cpw = csc // nw
    mesh = plsc.VectorSubcoreMesh(
        core_axis_name="c", subcore_axis_name="s",
        num_cores=_NC, num_subcores=_NS)

    @functools.partial(
        pl.kernel,
        mesh=mesh,
        out_type=jax.ShapeDtypeStruct((B, csc, HW), jnp.float32),
        scratch_types=[
            pltpu.VMEM((B * HW,), jnp.float32),
            pltpu.VMEM((B * HW,), jnp.int32),
            pltpu.SemaphoreType.DMA,
        ],
    )
    def sck(x_hbm, o_hbm, xv, sv, sem):
        wid = lax.axis_index("s") * _NC + lax.axis_index("c")

        def chan_body(ci, _):
            ch = wid * cpw + ci
            cps = [
                pltpu.async_copy(x_hbm.at[b, ch], xv.at[pl.ds(b * HW, HW)], sem)
                for b in range(B)
            ]
            for cp in cps:
                cp.wait()

            def key_fn(i, carry):
                idx = pl.ds(i * _L, _L)
                iv = lax.bitcast_convert_type(xv[idx], jnp.int32)
                sv[idx] = jnp.where(iv >= 0, iv, _splat_i32(_INT_MIN) - iv)
                return carry

            _foreach16(n16_ch, key_fn, 0)

            def map_body(m, _):
                base = m * n16_map

                K1 = _kth_key_sc(
                    lambda i: sv[pl.ds((base + i) * _L, _L)], n16_map, k1)

                def mask_fn(i, carry):
                    idx = pl.ds((base + i) * _L, _L)
                    s = sv[idx]
                    keep = s >= K1
                    xv[idx] = jnp.where(keep, xv[idx], 0.0)
                    sv[idx] = jnp.where(keep, s, _splat_i32(0))
                    return carry

                _foreach16(n16_map, mask_fn, 0)
                return 0

            lax.fori_loop(0, B, map_body, 0)

            K2 = _kth_key_sc(lambda i: sv[pl.ds(i * _L, _L)], n16_ch, k2)

            def mask2_fn(i, carry):
                idx = pl.ds(i * _L, _L)
                xv[idx] = jnp.where(sv[idx] >= K2, xv[idx], 0.0)
                return carry

            _foreach16(n16_ch, mask2_fn, 0)

            cps_o = [
                pltpu.async_copy(xv.at[pl.ds(b * HW, HW)], o_hbm.at[b, ch], sem)
                for b in range(B)
            ]
            for cp in cps_o:
                cp.wait()
            return 0

        lax.fori_loop(0, cpw, chan_body, 0)

    return sck


_C_SC = 96


def kernel(x):
    B, C, H, W = x.shape
    HW = H * W
    k1 = math.ceil(0.5 * H * W)
    k2 = math.ceil(0.5 * k1 * B)
    xr = x.reshape(B, C, HW)

    csc = min(_C_SC, C)
    csc -= csc % (_NC * _NS)
    parts = []
    if csc:
        parts.append(_make_sc_kernel(B, HW, csc, k1, k2)(xr[:, :csc]))
    if csc < C:
        parts.append(_tc_topk(xr[:, csc:], k1, k2))
    out = parts[0] if len(parts) == 1 else jnp.concatenate(parts, axis=1)
    return out.reshape(B, C, H, W)


# speedup vs baseline: 2.7784x; 1.1294x over previous
import functools
import math

import jax
import jax.numpy as jnp
from jax import lax
from jax.experimental import pallas as pl
from jax.experimental.pallas import tpu as pltpu
from jax.experimental.pallas import tpu_sc as plsc

_INT_MIN = -2147483648


def _skey(f):
    i = jax.lax.bitcast_convert_type(f, jnp.int32)
    return jnp.where(i >= 0, i, jnp.int32(_INT_MIN) - i)


def _kth_largest_key(skey, k, count_fn, row_shape):
    K = jnp.full(row_shape, _INT_MIN, dtype=jnp.int32)
    for b in range(31, -1, -1):
        addend = _INT_MIN if b == 31 else (1 << b)
        cand = K + jnp.int32(addend)
        cnt = count_fn(skey >= _bcast_row(cand, skey.shape))
        K = jnp.where(cnt >= k, cand, K)
    return K


def _bcast_row(row_val, full_shape):
    return jnp.broadcast_to(row_val[..., None], full_shape)


def _make_tc_kernel(k1, k2):
    def _kern(x_ref, o_ref):
        x = x_ref[...]
        s1 = _skey(x)

        def count1(ge):
            return jnp.sum(ge.astype(jnp.int32), axis=2)

        K1 = _kth_largest_key(s1, k1, count1, s1.shape[:2])
        keep1 = s1 >= _bcast_row(K1, s1.shape)
        m1 = jnp.where(keep1, x, 0.0)
        s2 = jnp.where(keep1, s1, 0)

        def count2(ge):
            c = jnp.sum(ge.astype(jnp.int32), axis=2)
            return jnp.sum(c, axis=0, keepdims=True)

        K2 = _kth_largest_key(s2, k2, count2, (1, s1.shape[1]))
        K2 = jnp.broadcast_to(K2, s1.shape[:2])
        o_ref[...] = jnp.where(s2 >= _bcast_row(K2, s1.shape), m1, 0.0)

    return _kern


def _tc_topk(xr, k1, k2, c_lo=0):
    B, C, HW = xr.shape
    n = C - c_lo
    cb = next(c for c in (64, 32, 16, 8, 4, 2, 1)
              if n % c == 0 and c_lo % c == 0)
    off = c_lo // cb
    return pl.pallas_call(
        _make_tc_kernel(k1, k2),
        grid=(n // cb,),
        in_specs=[pl.BlockSpec((B, cb, HW), lambda i: (0, i + off, 0))],
        out_specs=pl.BlockSpec((B, cb, HW), lambda i: (0, i + off, 0)),
        out_shape=jax.ShapeDtypeStruct((B, C, HW), jnp.float32),
    )(xr)


def _insert_channels(base, part):
    B, C, HW = base.shape
    csc = part.shape[1]
    cb = next(c for c in (64, 32, 16, 8, 4, 2, 1) if csc % c == 0)

    def _ins(_, p_ref, o_ref):
        o_ref[...] = p_ref[...]

    return pl.pallas_call(
        _ins,
        grid=(csc // cb,),
        in_specs=[pl.BlockSpec(memory_space=pl.ANY),
                  pl.BlockSpec((B, cb, HW), lambda i: (0, i, 0))],
        out_specs=pl.BlockSpec((B, cb, HW), lambda i: (0, i, 0)),
        out_shape=jax.ShapeDtypeStruct((B, C, HW), jnp.float32),
        input_output_aliases={0: 0},
    )(base, part)


_NC, _NS, _L = 2, 16, 16


def _splat_i32(x):
    return jnp.broadcast_to(jnp.asarray(x, jnp.int32), (_L,))


def _foreach16(n16, fn, init):
    unroll = next(u for u in (8, 4, 2, 1) if n16 % u == 0)

    def body(j, carry):
        for u in range(unroll):
            carry = fn(j * unroll + u, carry)
        return carry

    return lax.fori_loop(0, n16 // unroll, body, init)


def _lane_sum(cnt):
    total = cnt[0]
    for i in range(1, _L):
        total = total + cnt[i]
    return total


def _kth_key_sc(read_slice, n16, k):
    def bit_body(bi, K):
        shift = jnp.broadcast_to(31 - bi, (_L,)).astype(jnp.int32)
        cand = K + (_splat_i32(1) << shift)

        def cnt_fn(i, cnt):
            ge = read_slice(i) >= cand
            return cnt + jnp.where(ge, _splat_i32(1), _splat_i32(0))

        cnt = _foreach16(n16, cnt_fn, _splat_i32(0))
        return jnp.where(_lane_sum(cnt) >= k, cand, K)

    return lax.fori_loop(0, 32, bit_body, _splat_i32(_INT_MIN))


def _make_sc_kernel(B, HW, csc, k1, k2):
    n16_map = HW // _L
    n16_ch = (B * HW) // _L
    nw = _NC * _NS
    cpw = csc // nw
    mesh = plsc.VectorSubcoreMesh(
        core_axis_name="c", subcore_axis_name="s",
        num_cores=_NC, num_subcores=_NS)

    @functools.partial(
        pl.kernel,
        mesh=mesh,
        out_type=jax.ShapeDtypeStruct((B, csc, HW), jnp.float32),
        scratch_types=[
            pltpu.VMEM((B * HW,), jnp.float32),
            pltpu.VMEM((B * HW,), jnp.int32),
            pltpu.SemaphoreType.DMA,
        ],
    )
    def sck(x_hbm, o_hbm, xv, sv, sem):
        wid = lax.axis_index("s") * _NC + lax.axis_index("c")

        def chan_body(ci, _):
            ch = wid * cpw + ci
            cps = [
                pltpu.async_copy(x_hbm.at[b, ch], xv.at[pl.ds(b * HW, HW)], sem)
                for b in range(B)
            ]
            for cp in cps:
                cp.wait()

            def key_fn(i, carry):
                idx = pl.ds(i * _L, _L)
                iv = lax.bitcast_convert_type(xv[idx], jnp.int32)
                sv[idx] = jnp.where(iv >= 0, iv, _splat_i32(_INT_MIN) - iv)
                return carry

            _foreach16(n16_ch, key_fn, 0)

            def map_body(m, _):
                base = m * n16_map

                K1 = _kth_key_sc(
                    lambda i: sv[pl.ds((base + i) * _L, _L)], n16_map, k1)

                def mask_fn(i, carry):
                    idx = pl.ds((base + i) * _L, _L)
                    s = sv[idx]
                    keep = s >= K1
                    xv[idx] = jnp.where(keep, xv[idx], 0.0)
                    sv[idx] = jnp.where(keep, s, _splat_i32(0))
                    return carry

                _foreach16(n16_map, mask_fn, 0)
                return 0

            lax.fori_loop(0, B, map_body, 0)

            K2 = _kth_key_sc(lambda i: sv[pl.ds(i * _L, _L)], n16_ch, k2)

            def mask2_fn(i, carry):
                idx = pl.ds(i * _L, _L)
                xv[idx] = jnp.where(sv[idx] >= K2, xv[idx], 0.0)
                return carry

            _foreach16(n16_ch, mask2_fn, 0)

            cps_o = [
                pltpu.async_copy(xv.at[pl.ds(b * HW, HW)], o_hbm.at[b, ch], sem)
                for b in range(B)
            ]
            for cp in cps_o:
                cp.wait()
            return 0

        lax.fori_loop(0, cpw, chan_body, 0)

    return sck


_C_SC = 96


def kernel(x):
    B, C, H, W = x.shape
    HW = H * W
    k1 = math.ceil(0.5 * H * W)
    k2 = math.ceil(0.5 * k1 * B)
    xr = x.reshape(B, C, HW)

    csc = min(_C_SC, C)
    csc -= csc % (_NC * _NS)
    if csc == C:
        out = _make_sc_kernel(B, HW, csc, k1, k2)(xr)
    elif csc == 0:
        out = _tc_topk(xr, k1, k2)
    else:
        sc_out = _make_sc_kernel(B, HW, csc, k1, k2)(xr)
        tc_out = _tc_topk(xr, k1, k2, c_lo=csc)
        out = _insert_channels(tc_out, sc_out)
    return out.reshape(B, C, H, W)
